# Initial kernel scaffold; baseline (speedup 1.0000x reference)
import jax
import jax.numpy as jnp
from jax.experimental import pallas as pl


def kernel(input_context, cand_idx, pos_table, W):
    raise NotImplementedError("write your pallas kernel here")


# speedup vs baseline: 10.4230x; 10.4230x over previous
import functools

import jax
import jax.numpy as jnp
from jax.experimental import pallas as pl

B, L, K = 1024, 50, 9
D_IN, D_OUT, P = 128, 200, 150
T = 512


def _attn_block(x_ref, idx_ref, tab_ref, w_ref, out_ref):
    x = x_ref[...]
    idx = idx_ref[...]
    tab = tab_ref[...]
    w = w_ref[...]

    lin = jax.lax.dot_general(x, w, (((1,), (1,)), ((), ())),
                              preferred_element_type=jnp.float32)
    s = jax.lax.dot_general(lin, tab, (((1,), (1,)), ((), ())),
                            preferred_element_type=jnp.float32)

    iota = jax.lax.broadcasted_iota(jnp.int32, (T, P), 1)
    cnt = jnp.zeros((T, P), jnp.float32)
    for k in range(K):
        cnt += (idx[:, k:k + 1] == iota).astype(jnp.float32)
    valid = cnt > 0.0

    m = jnp.max(jnp.where(valid, s, -jnp.inf), axis=1, keepdims=True)
    e = jnp.where(valid, jnp.exp(s - m), 0.0) * cnt
    probs = e / jnp.sum(e, axis=1, keepdims=True)

    out_ref[...] = jax.lax.dot_general(
        probs, tab, (((1,), (0,)), ((), ())),
        preferred_element_type=jnp.float32)


@jax.jit
def kernel(input_context, cand_idx, pos_table, W):
    n = B * L
    x = input_context.reshape(n, D_IN)
    idx = cand_idx.reshape(n, K).astype(jnp.int32)
    grid = (n // T,)
    out = pl.pallas_call(
        _attn_block,
        grid=grid,
        in_specs=[
            pl.BlockSpec((T, D_IN), lambda i: (i, 0)),
            pl.BlockSpec((T, K), lambda i: (i, 0)),
            pl.BlockSpec((P, D_OUT), lambda i: (0, 0)),
            pl.BlockSpec((D_OUT, D_IN), lambda i: (0, 0)),
        ],
        out_specs=pl.BlockSpec((T, D_OUT), lambda i: (i, 0)),
        out_shape=jax.ShapeDtypeStruct((n, D_OUT), jnp.float32),
    )(x, idx, pos_table, W)
    return out.reshape(B, L, D_OUT)
